# Initial kernel scaffold; baseline (speedup 1.0000x reference)
import jax
import jax.numpy as jnp
from jax.experimental import pallas as pl


def kernel(x):
    raise NotImplementedError("write your pallas kernel here")


# speedup vs baseline: 1.0136x; 1.0136x over previous
import jax
import jax.numpy as jnp
from jax.experimental import pallas as pl

_OBJECT_PROBS = (0.1, 0.1, 0.1, 0.1, 0.15, 0.15, 0.05, 0.05)
_COLS_PER_OBJ = 32
_MASK_VALUE = 0.0


def _mask_bits(batch):
    rng = jax.random.key(42)
    bits = jnp.zeros((batch,), jnp.int32)
    for i, p in enumerate(_OBJECT_PROBS):
        k = jax.random.fold_in(rng, i)
        m = jax.random.uniform(k, (batch,)) < p
        bits = bits | (m.astype(jnp.int32) << i)
    return bits


def _mask_kernel(bits_ref, x_ref, o_ref):
    x = x_ref[...]
    bits = bits_ref[...]
    obj = jax.lax.broadcasted_iota(jnp.int32, x.shape, 1) // _COLS_PER_OBJ
    masked = (jnp.right_shift(bits, obj) & 1) != 0
    o_ref[...] = jnp.where(masked, jnp.float32(_MASK_VALUE), x)


def kernel(x):
    b, f = x.shape
    bits = _mask_bits(b).reshape(b, 1)
    rows = 1024
    return pl.pallas_call(
        _mask_kernel,
        grid=(b // rows,),
        in_specs=[
            pl.BlockSpec((rows, 1), lambda i: (i, 0)),
            pl.BlockSpec((rows, f), lambda i: (i, 0)),
        ],
        out_specs=pl.BlockSpec((rows, f), lambda i: (i, 0)),
        out_shape=jax.ShapeDtypeStruct((b, f), x.dtype),
    )(bits, x)


# speedup vs baseline: 1.0843x; 1.0697x over previous
import jax
import jax.numpy as jnp
from jax.experimental import pallas as pl
from jax.experimental.pallas import tpu as pltpu

_OBJECT_PROBS = (0.1, 0.1, 0.1, 0.1, 0.15, 0.15, 0.05, 0.05)
_COLS_PER_OBJ = 32
_MASK_VALUE = 0.0


def _mask_bits(batch):
    rng = jax.random.key(42)
    bits = jnp.zeros((batch,), jnp.int32)
    for i, p in enumerate(_OBJECT_PROBS):
        k = jax.random.fold_in(rng, i)
        m = jax.random.uniform(k, (batch,)) < p
        bits = bits | (m.astype(jnp.int32) << i)
    return bits


def _mask_kernel(bits_ref, x_ref, o_ref):
    x = x_ref[...]
    bits = bits_ref[...]
    obj = jax.lax.broadcasted_iota(jnp.int32, x.shape, 1) // _COLS_PER_OBJ
    masked = (jnp.right_shift(bits, obj) & 1) != 0
    o_ref[...] = jnp.where(masked, jnp.float32(_MASK_VALUE), x)


def kernel(x):
    b, f = x.shape
    bits = _mask_bits(b).reshape(b, 1)
    rows = 2048
    return pl.pallas_call(
        _mask_kernel,
        grid=(b // rows,),
        in_specs=[
            pl.BlockSpec((rows, 1), lambda i: (i, 0)),
            pl.BlockSpec((rows, f), lambda i: (i, 0)),
        ],
        out_specs=pl.BlockSpec((rows, f), lambda i: (i, 0)),
        out_shape=jax.ShapeDtypeStruct((b, f), x.dtype),
        compiler_params=pltpu.CompilerParams(
            dimension_semantics=("parallel",),
        ),
    )(bits, x)


# speedup vs baseline: 1.1135x; 1.0269x over previous
import jax
import jax.numpy as jnp
from jax.experimental import pallas as pl
from jax.experimental.pallas import tpu as pltpu

_OBJECT_PROBS = (0.1, 0.1, 0.1, 0.1, 0.15, 0.15, 0.05, 0.05)
_COLS_PER_OBJ = 32
_MASK_VALUE = 0.0


def _mask_bits(batch):
    rng = jax.random.key(42)
    bits = jnp.zeros((batch,), jnp.int32)
    for i, p in enumerate(_OBJECT_PROBS):
        k = jax.random.fold_in(rng, i)
        m = jax.random.uniform(k, (batch,)) < p
        bits = bits | (m.astype(jnp.int32) << i)
    return bits


def _mask_kernel(bits_ref, x_ref, o_ref):
    x = x_ref[...]
    bits = bits_ref[...]
    obj = jax.lax.broadcasted_iota(jnp.int32, x.shape, 1) // _COLS_PER_OBJ
    masked = (jnp.right_shift(bits, obj) & 1) != 0
    o_ref[...] = jnp.where(masked, jnp.float32(_MASK_VALUE), x)


def kernel(x):
    b, f = x.shape
    bits = _mask_bits(b).reshape(b, 1)
    rows = 4096
    return pl.pallas_call(
        _mask_kernel,
        grid=(b // rows,),
        in_specs=[
            pl.BlockSpec((rows, 1), lambda i: (i, 0)),
            pl.BlockSpec((rows, f), lambda i: (i, 0)),
        ],
        out_specs=pl.BlockSpec((rows, f), lambda i: (i, 0)),
        out_shape=jax.ShapeDtypeStruct((b, f), x.dtype),
        compiler_params=pltpu.CompilerParams(
            dimension_semantics=("parallel",),
        ),
    )(bits, x)


# speedup vs baseline: 1.1234x; 1.0089x over previous
import jax
import jax.numpy as jnp
from jax.experimental import pallas as pl
from jax.experimental.pallas import tpu as pltpu

_OBJECT_PROBS = (0.1, 0.1, 0.1, 0.1, 0.15, 0.15, 0.05, 0.05)
_COLS_PER_OBJ = 32
_MASK_VALUE = 0.0


def _mask_bits(batch):
    rng = jax.random.key(42)
    bits = jnp.zeros((batch,), jnp.int32)
    for i, p in enumerate(_OBJECT_PROBS):
        k = jax.random.fold_in(rng, i)
        m = jax.random.uniform(k, (batch,)) < p
        bits = bits | (m.astype(jnp.int32) << i)
    return bits


def _mask_kernel(bits_ref, x_ref, o_ref):
    x = x_ref[...]
    bits = bits_ref[...]
    obj = jax.lax.broadcasted_iota(jnp.int32, x.shape, 1) // _COLS_PER_OBJ
    masked = (jnp.right_shift(bits, obj) & 1) != 0
    o_ref[...] = jnp.where(masked, jnp.float32(_MASK_VALUE), x)


def kernel(x):
    b, f = x.shape
    bits = _mask_bits(b).reshape(b, 1)
    rows = 8192
    return pl.pallas_call(
        _mask_kernel,
        grid=(b // rows,),
        in_specs=[
            pl.BlockSpec((rows, 1), lambda i: (i, 0)),
            pl.BlockSpec((rows, f), lambda i: (i, 0)),
        ],
        out_specs=pl.BlockSpec((rows, f), lambda i: (i, 0)),
        out_shape=jax.ShapeDtypeStruct((b, f), x.dtype),
        compiler_params=pltpu.CompilerParams(
            dimension_semantics=("parallel",),
        ),
    )(bits, x)
